# baseline (speedup 1.0000x reference)
import jax
import jax.numpy as jnp
from jax.experimental import pallas as pl
from jax.experimental.pallas import tpu as pltpu

_G = 512
_BR = 1280
_K = 128


def _pass_a(lo_ref, hi_ref, x_ref, bc_ref, br_ref, w1_ref, w2_ref, y_ref,
            sum_ref, max_ref):
    i = pl.program_id(0)

    @pl.when(i == 0)
    def _init():
        sum_ref[...] = jnp.zeros_like(sum_ref)
        max_ref[...] = jnp.full_like(max_ref, -jnp.inf)

    b = bc_ref[0, :, :]
    brow = br_ref[0, :, :]
    x = x_ref[...]
    s_lo = lo_ref[i]
    s_hi = hi_ref[i]

    kio = jax.lax.broadcasted_iota(jnp.int32, (_K, _BR), 0)
    mt = (kio == (brow - s_lo)).astype(jnp.float32)
    part = jax.lax.dot_general(mt, x, (((1,), (0,)), ((), ())),
                               precision=jax.lax.Precision.HIGHEST,
                               preferred_element_type=jnp.float32)
    sum_ref[pl.ds(s_lo, _K), :] = sum_ref[pl.ds(s_lo, _K), :] + part

    def mbody(s, carry):
        m = b == s
        mx = jnp.max(jnp.where(m, x, -jnp.inf), axis=0, keepdims=True)
        max_ref[pl.ds(s, 1), :] = jnp.maximum(max_ref[pl.ds(s, 1), :], mx)
        return carry

    jax.lax.fori_loop(s_lo, s_hi + 1, mbody, 0)

    def sbody(s, carry):
        m = b == s
        sm = jnp.sum(jnp.where(m, x, 0.0), axis=0, keepdims=True)
        sum_ref[pl.ds(s, 1), :] = sum_ref[pl.ds(s, 1), :] + sm
        return carry

    jax.lax.fori_loop(s_lo + _K, s_hi + 1, sbody, 0)

    @pl.when(i == pl.num_programs(0) - 1)
    def _finish():
        mx = max_ref[0:_G, :]
        mx = jnp.where(mx == -jnp.inf, 0.0, mx)
        sm = sum_ref[0:_G, :]
        w1 = w1_ref[...]
        w2 = w2_ref[...]
        h1 = jnp.maximum(jnp.dot(mx, w1, preferred_element_type=jnp.float32), 0.0)
        o1 = jnp.dot(h1, w2, preferred_element_type=jnp.float32)
        h2 = jnp.maximum(jnp.dot(sm, w1, preferred_element_type=jnp.float32), 0.0)
        o2 = jnp.dot(h2, w2, preferred_element_type=jnp.float32)
        y_ref[...] = jnp.maximum(o1 + o2, 0.0)


def _pass_b(lo_ref, hi_ref, x_ref, bc_ref, y_ref, o_ref):
    i = pl.program_id(0)
    b = bc_ref[0, :, :]
    s_lo = lo_ref[i]
    s_hi = hi_ref[i]

    kio = jax.lax.broadcasted_iota(jnp.int32, (_BR, _K), 1)
    m = (kio == (b - s_lo)).astype(jnp.float32)
    ys = y_ref[pl.ds(s_lo, _K), :]
    rows = jax.lax.dot_general(m, ys, (((1,), (0,)), ((), ())),
                               precision=jax.lax.Precision.HIGHEST,
                               preferred_element_type=jnp.float32)
    o_ref[...] = rows

    def body(s, carry):
        yy = y_ref[pl.ds(s, 1), :]
        mm = b == s
        o_ref[...] = jnp.where(mm, yy, o_ref[...])
        return carry

    jax.lax.fori_loop(s_lo + _K, s_hi + 1, body, 0)
    o_ref[...] = o_ref[...] * x_ref[...]


def kernel(x, batch, W1, W2):
    n, c = x.shape
    nb = n // _BR
    bcol = batch.reshape(nb, _BR, 1)
    brow = batch.reshape(nb, 1, _BR)
    blo = bcol[:, 0, 0]
    bhi = bcol[:, _BR - 1, 0]

    y = pl.pallas_call(
        _pass_a,
        grid=(nb,),
        in_specs=[
            pl.BlockSpec(memory_space=pltpu.SMEM),
            pl.BlockSpec(memory_space=pltpu.SMEM),
            pl.BlockSpec((_BR, c), lambda i: (i, 0)),
            pl.BlockSpec((1, _BR, 1), lambda i: (i, 0, 0)),
            pl.BlockSpec((1, 1, _BR), lambda i: (i, 0, 0)),
            pl.BlockSpec((c, c // 8), lambda i: (0, 0)),
            pl.BlockSpec((c // 8, c), lambda i: (0, 0)),
        ],
        out_specs=pl.BlockSpec((_G, c), lambda i: (0, 0)),
        out_shape=jax.ShapeDtypeStruct((_G, c), jnp.float32),
        scratch_shapes=[
            pltpu.VMEM((_G + _K, c), jnp.float32),
            pltpu.VMEM((_G + _K, c), jnp.float32),
        ],
        compiler_params=pltpu.CompilerParams(
            dimension_semantics=("arbitrary",),
        ),
    )(blo, bhi, x, bcol, brow, W1, W2)

    ypad = jnp.concatenate([y, jnp.zeros((_K, c), jnp.float32)], axis=0)

    out = pl.pallas_call(
        _pass_b,
        grid=(nb,),
        in_specs=[
            pl.BlockSpec(memory_space=pltpu.SMEM),
            pl.BlockSpec(memory_space=pltpu.SMEM),
            pl.BlockSpec((_BR, c), lambda i: (i, 0)),
            pl.BlockSpec((1, _BR, 1), lambda i: (i, 0, 0)),
            pl.BlockSpec((_G + _K, c), lambda i: (0, 0)),
        ],
        out_specs=pl.BlockSpec((_BR, c), lambda i: (i, 0)),
        out_shape=jax.ShapeDtypeStruct((n, c), jnp.float32),
        compiler_params=pltpu.CompilerParams(
            dimension_semantics=("arbitrary",),
        ),
    )(blo, bhi, x, bcol, ypad)
    return out


# speedup vs baseline: 1.1645x; 1.1645x over previous
import jax
import jax.numpy as jnp
from jax.experimental import pallas as pl
from jax.experimental.pallas import tpu as pltpu

_G = 512
_BR = 1280
_K = 128


def _pass_a(lo_ref, hi_ref, x_ref, bc_ref, br_ref, w1_ref, w2_ref, y_ref,
            sum_ref, max_ref):
    i = pl.program_id(0)

    @pl.when(i == 0)
    def _init():
        sum_ref[...] = jnp.zeros_like(sum_ref)
        max_ref[...] = jnp.full_like(max_ref, -jnp.inf)

    b = bc_ref[0, :, :]
    brow = br_ref[0, :, :]
    x = x_ref[...]
    s_lo = lo_ref[i]
    s_hi = hi_ref[i]

    kio = jax.lax.broadcasted_iota(jnp.int32, (_K, _BR), 0)
    mt = (kio == (brow - s_lo)).astype(jnp.bfloat16)
    xhi = x.astype(jnp.bfloat16)
    xlo = (x - xhi.astype(jnp.float32)).astype(jnp.bfloat16)
    dn = (((1,), (0,)), ((), ()))
    part = (jax.lax.dot_general(mt, xhi, dn, preferred_element_type=jnp.float32)
            + jax.lax.dot_general(mt, xlo, dn, preferred_element_type=jnp.float32))
    sum_ref[pl.ds(s_lo, _K), :] = sum_ref[pl.ds(s_lo, _K), :] + part

    def mbody(s, carry):
        m = b == s
        mx = jnp.max(jnp.where(m, x, -jnp.inf), axis=0, keepdims=True)
        max_ref[pl.ds(s, 1), :] = jnp.maximum(max_ref[pl.ds(s, 1), :], mx)
        return carry

    jax.lax.fori_loop(s_lo, s_hi + 1, mbody, 0)

    def sbody(s, carry):
        m = b == s
        sm = jnp.sum(jnp.where(m, x, 0.0), axis=0, keepdims=True)
        sum_ref[pl.ds(s, 1), :] = sum_ref[pl.ds(s, 1), :] + sm
        return carry

    jax.lax.fori_loop(s_lo + _K, s_hi + 1, sbody, 0)

    @pl.when(i == pl.num_programs(0) - 1)
    def _finish():
        mx = max_ref[0:_G, :]
        mx = jnp.where(mx == -jnp.inf, 0.0, mx)
        sm = sum_ref[0:_G, :]
        w1 = w1_ref[...]
        w2 = w2_ref[...]
        h1 = jnp.maximum(jnp.dot(mx, w1, preferred_element_type=jnp.float32), 0.0)
        o1 = jnp.dot(h1, w2, preferred_element_type=jnp.float32)
        h2 = jnp.maximum(jnp.dot(sm, w1, preferred_element_type=jnp.float32), 0.0)
        o2 = jnp.dot(h2, w2, preferred_element_type=jnp.float32)
        y_ref[...] = jnp.maximum(o1 + o2, 0.0)


def _pass_b(lo_ref, hi_ref, x_ref, bc_ref, y_ref, o_ref):
    i = pl.program_id(0)
    b = bc_ref[0, :, :]
    s_lo = lo_ref[i]
    s_hi = hi_ref[i]

    kio = jax.lax.broadcasted_iota(jnp.int32, (_BR, _K), 1)
    m = (kio == (b - s_lo)).astype(jnp.float32)
    ys = y_ref[pl.ds(s_lo, _K), :]
    rows = jax.lax.dot_general(m, ys, (((1,), (0,)), ((), ())),
                               preferred_element_type=jnp.float32)
    o_ref[...] = rows

    def body(s, carry):
        yy = y_ref[pl.ds(s, 1), :]
        mm = b == s
        o_ref[...] = jnp.where(mm, yy, o_ref[...])
        return carry

    jax.lax.fori_loop(s_lo + _K, s_hi + 1, body, 0)
    o_ref[...] = o_ref[...] * x_ref[...]


def kernel(x, batch, W1, W2):
    n, c = x.shape
    nb = n // _BR
    bcol = batch.reshape(nb, _BR, 1)
    brow = batch.reshape(nb, 1, _BR)
    blo = bcol[:, 0, 0]
    bhi = bcol[:, _BR - 1, 0]

    y = pl.pallas_call(
        _pass_a,
        grid=(nb,),
        in_specs=[
            pl.BlockSpec(memory_space=pltpu.SMEM),
            pl.BlockSpec(memory_space=pltpu.SMEM),
            pl.BlockSpec((_BR, c), lambda i: (i, 0)),
            pl.BlockSpec((1, _BR, 1), lambda i: (i, 0, 0)),
            pl.BlockSpec((1, 1, _BR), lambda i: (i, 0, 0)),
            pl.BlockSpec((c, c // 8), lambda i: (0, 0)),
            pl.BlockSpec((c // 8, c), lambda i: (0, 0)),
        ],
        out_specs=pl.BlockSpec((_G, c), lambda i: (0, 0)),
        out_shape=jax.ShapeDtypeStruct((_G, c), jnp.float32),
        scratch_shapes=[
            pltpu.VMEM((_G + _K, c), jnp.float32),
            pltpu.VMEM((_G + _K, c), jnp.float32),
        ],
        compiler_params=pltpu.CompilerParams(
            dimension_semantics=("arbitrary",),
        ),
    )(blo, bhi, x, bcol, brow, W1, W2)

    ypad = jnp.concatenate([y, jnp.zeros((_K, c), jnp.float32)], axis=0)

    out = pl.pallas_call(
        _pass_b,
        grid=(nb,),
        in_specs=[
            pl.BlockSpec(memory_space=pltpu.SMEM),
            pl.BlockSpec(memory_space=pltpu.SMEM),
            pl.BlockSpec((_BR, c), lambda i: (i, 0)),
            pl.BlockSpec((1, _BR, 1), lambda i: (i, 0, 0)),
            pl.BlockSpec((_G + _K, c), lambda i: (0, 0)),
        ],
        out_specs=pl.BlockSpec((_BR, c), lambda i: (i, 0)),
        out_shape=jax.ShapeDtypeStruct((n, c), jnp.float32),
        compiler_params=pltpu.CompilerParams(
            dimension_semantics=("arbitrary",),
        ),
    )(blo, bhi, x, bcol, ypad)
    return out
